# baseline (speedup 1.0000x reference)
import functools

import numpy as np
import jax
import jax.numpy as jnp
from jax import lax
from jax.experimental import pallas as pl
from jax.experimental.pallas import tpu as pltpu
from jax.experimental.pallas import tpu_sc as plsc

NN = 10000
NE = 320000
HID = 128

_F32 = jnp.float32
_pc = pl.pallas_call
EB = 2000


def _dot(a, b):
    return jnp.dot(a, b, preferred_element_type=_F32)


def _bn_full(x, g, b):
    m = jnp.mean(x, 0, keepdims=True)
    v = jnp.mean((x - m) ** 2, 0, keepdims=True)
    return (x - m) / jnp.sqrt(v + 1e-5) * g + b


def _r1(v):
    return v.reshape(1, -1)


def _node_head(h_V, p):
    def body(hv, nW, nb, nng, nnb, w1, b1, g1, bb1, w2, b2, g2, bb2, w3, b3, out):
        y = _dot(hv[...], nW[...]) + nb[...]
        y = _bn_full(y, nng[...], nnb[...])
        y = _dot(y, w1[...]) + b1[...]
        y = jnp.where(y >= 0, y, 0.01 * y)
        y = _bn_full(y, g1[...], bb1[...])
        y = _dot(y, w2[...]) + b2[...]
        y = jnp.where(y >= 0, y, 0.01 * y)
        y = _bn_full(y, g2[...], bb2[...])
        out[...] = _dot(y, w3[...]) + b3[...]

    hvp = jnp.pad(h_V, ((0, 0), (0, 4)))
    nWp = jnp.pad(p['node_W'], ((0, 4), (0, 0)))
    return _pc(body, out_shape=jax.ShapeDtypeStruct((NN, HID), _F32))(
        hvp, nWp, _r1(p['node_b']), _r1(p['nn_g']), _r1(p['nn_b']),
        p['wv1_W'], _r1(p['wv1_b']), _r1(p['wbn1_g']), _r1(p['wbn1_b']),
        p['wv2_W'], _r1(p['wv2_b']), _r1(p['wbn2_g']), _r1(p['wbn2_b']),
        p['wv3_W'], _r1(p['wv3_b']))


def _edge_stats(h_Pp, Wp, b):
    nblk = NE // EB

    def body(hp, W, bb, out, acc):
        i = pl.program_id(0)
        y = _dot(hp[...], W[...]) + bb[...]

        @pl.when(i == 0)
        def _():
            acc[...] = jnp.zeros_like(acc)

        acc[0:1, :] = acc[0:1, :] + jnp.sum(y, 0, keepdims=True)
        acc[1:2, :] = acc[1:2, :] + jnp.sum(y * y, 0, keepdims=True)
        out[...] = acc[...]

    return _pc(
        body,
        grid=(nblk,),
        in_specs=[pl.BlockSpec((EB, 24), lambda i: (i, 0)),
                  pl.BlockSpec((24, 128), lambda i: (0, 0)),
                  pl.BlockSpec((1, 128), lambda i: (0, 0))],
        out_specs=pl.BlockSpec((2, 128), lambda i: (0, 0)),
        out_shape=jax.ShapeDtypeStruct((2, 128), _F32),
        scratch_shapes=[pltpu.VMEM((2, 128), _F32)],
    )(h_Pp, Wp, b)


def _edge_embed(h_Pp, stats, Wp, b, g, bb, weW, web):
    nblk = NE // EB

    def body(hp, st, W, b_, g_, bb_, wW, wb, out):
        y = _dot(hp[...], W[...]) + b_[...]
        sa = st[...]
        m = sa[0:1, :] * np.float32(1.0 / NE)
        v = sa[1:2, :] * np.float32(1.0 / NE) - m * m
        y = (y - m) / jnp.sqrt(v + 1e-5) * g_[...] + bb_[...]
        out[...] = _dot(y, wW[...]) + wb[...]

    return _pc(
        body,
        grid=(nblk,),
        in_specs=[pl.BlockSpec((EB, 24), lambda i: (i, 0)),
                  pl.BlockSpec((2, 128), lambda i: (0, 0)),
                  pl.BlockSpec((24, 128), lambda i: (0, 0)),
                  pl.BlockSpec((1, 128), lambda i: (0, 0)),
                  pl.BlockSpec((1, 128), lambda i: (0, 0)),
                  pl.BlockSpec((1, 128), lambda i: (0, 0)),
                  pl.BlockSpec((128, 128), lambda i: (0, 0)),
                  pl.BlockSpec((1, 128), lambda i: (0, 0))],
        out_specs=pl.BlockSpec((EB, 128), lambda i: (i, 0)),
        out_shape=jax.ShapeDtypeStruct((NE, 128), _F32),
    )(h_Pp, stats, Wp, b, g, bb, weW, web)


def _edge_attn(g1, g2, hP, lp, blk0):
    B1 = lp['B1_W']
    B3p = jnp.pad(lp['B3_W'], ((0, 0), (0, 12)))
    b3p = _r1(jnp.pad(lp['B3_b'], (0, 12)))
    WV = lp['WV']
    sq = np.float32(np.sqrt(32.0))
    nblk = NE2 // EB

    def body(g1r, g2r, hpr, a_, b_, c_, b1r, B2r, b2r, B3r, b3r, wva, wvb,
             o128, o8):
        x1 = g1r[...]
        x2 = g2r[...]
        xp = hpr[...]
        h = _dot(x1, a_[...]) + _dot(xp, b_[...]) + _dot(x2, c_[...]) + b1r[...]
        h = jnp.maximum(h, 0.0)
        h = jnp.maximum(_dot(h, B2r[...]) + b2r[...], 0.0)
        w = (_dot(h, B3r[...]) + b3r[...]) / sq
        ex = jnp.exp(w)
        V = _dot(xp, wva[...]) + _dot(x2, wvb[...])
        row = lax.broadcasted_iota(jnp.int32, (16, 128), 0)
        col = lax.broadcasted_iota(jnp.int32, (16, 128), 1)
        R = jnp.where((col // 32 == row) & (row < 4), 1.0, 0.0).astype(_F32)
        E = _dot(ex, R)
        o128[...] = V * E
        o8[...] = E

    wspec = lambda shape: pl.BlockSpec(shape, lambda i: (0, 0))
    return _pc(
        body,
        grid=(nblk,),
        in_specs=[pl.BlockSpec((EB, 128), lambda i: (i, 0))] * 2 + [
            pl.BlockSpec((EB, 128), lambda i: (i + blk0, 0))] + [
            wspec((128, 128)), wspec((128, 128)), wspec((128, 128)),
            wspec((1, 128)), wspec((128, 128)), wspec((1, 128)),
            wspec((128, 16)), wspec((1, 16)),
            wspec((128, 128)), wspec((128, 128))],
        out_specs=[pl.BlockSpec((EB, 128), lambda i: (i, 0)),
                   pl.BlockSpec((EB, 128), lambda i: (i, 0))],
        out_shape=[jax.ShapeDtypeStruct((NE2, 128), _F32),
                   jax.ShapeDtypeStruct((NE2, 128), _F32)],
    )(g1, g2, hP,
      B1[0:128], B1[128:256], B1[256:384], _r1(lp['B1_b']),
      lp['B2_W'], _r1(lp['B2_b']), B3p, b3p, WV[0:128], WV[128:256])


_MESH = plsc.VectorSubcoreMesh(core_axis_name="c", subcore_axis_name="s")
GC = 40
NE2 = NE // 2
GPW2 = NE2 // 16
GNIT2 = GPW2 // GC


def _sc_gather(table, cid, src, eoff):
    @functools.partial(
        pl.kernel,
        out_type=(jax.ShapeDtypeStruct((NE2, 128), _F32),
                  jax.ShapeDtypeStruct((NE2, 128), _F32)),
        mesh=_MESH,
        scratch_types=[pltpu.VMEM((2, GC), jnp.int32),
                       pltpu.VMEM((2, GC, 128), _F32),
                       pltpu.SemaphoreType.DMA, pltpu.SemaphoreType.DMA,
                       pltpu.SemaphoreType.DMA, pltpu.SemaphoreType.DMA,
                       pltpu.SemaphoreType.DMA, pltpu.SemaphoreType.DMA],
    )
    def k(tab_hbm, cid_hbm, src_hbm, o1_hbm, o2_hbm, idx_v, rows_v,
          si0, si1, sg0, sg1, sw0, sw1):
        c = lax.axis_index("c")
        s = lax.axis_index("s")
        base = s * GPW2
        si = (si0, si1)
        sg = (sg0, sg1)
        sw = (sw0, sw1)

        def run(idx_hbm, out_hbm):
            for b in range(2):
                pltpu.async_copy(idx_hbm.at[pl.ds(eoff + base + b * GC, GC)],
                                 idx_v.at[b], si[b])

            def body(it, carry):
                for b in range(2):
                    i = it * 2 + b
                    off = base + i * GC
                    pltpu.make_async_copy(idx_hbm.at[pl.ds(eoff + off, GC)],
                                          idx_v.at[b], si[b]).wait()

                    @pl.when(it > 0)
                    def _():
                        pltpu.make_async_copy(
                            rows_v.at[b],
                            out_hbm.at[pl.ds(off - 2 * GC, GC)], sw[b]).wait()

                    pltpu.async_copy(tab_hbm.at[idx_v.at[b]], rows_v.at[b],
                                     sg[b]).wait()
                    pltpu.async_copy(rows_v.at[b], out_hbm.at[pl.ds(off, GC)],
                                     sw[b])

                    @pl.when(i + 2 < GNIT2)
                    def _():
                        pltpu.async_copy(
                            idx_hbm.at[pl.ds(eoff + off + 2 * GC, GC)],
                            idx_v.at[b], si[b])
                return carry

            lax.fori_loop(0, GNIT2 // 2, body, 0)
            for b in range(2):
                off = base + (GNIT2 - 2 + b) * GC
                pltpu.make_async_copy(rows_v.at[b], out_hbm.at[pl.ds(off, GC)],
                                      sw[b]).wait()

        @pl.when(c == 0)
        def _():
            run(cid_hbm, o1_hbm)

        @pl.when(c == 1)
        def _():
            run(src_hbm, o2_hbm)

    return k(table, cid, src)


def _sc_scatter(S128, E128, cid, z128, eoff):
    @functools.partial(
        pl.kernel,
        out_type=jax.ShapeDtypeStruct((2 * NN, 128), _F32),
        mesh=_MESH,
        scratch_types=[pltpu.VMEM((2, GC), jnp.int32),
                       pltpu.VMEM((2, GC, 128), _F32),
                       pltpu.SemaphoreType.DMA, pltpu.SemaphoreType.DMA,
                       pltpu.SemaphoreType.DMA, pltpu.SemaphoreType.DMA,
                       pltpu.VMEM_SHARED((NN, 128), _F32)],
    )
    def k(s128_hbm, e128_hbm, cid_hbm, z128_hbm, o_hbm, idx_v, buf_v,
          si0, si1, sp0, sp1, acc):
        c = lax.axis_index("c")
        s = lax.axis_index("s")
        r0 = s * 640
        si = (si0, si1)
        sp = (sp0, sp1)

        @pl.when(s < 15)
        def _():
            pltpu.sync_copy(z128_hbm.at[pl.ds(r0, 640)], acc.at[pl.ds(r0, 640)])

        @pl.when(s == 15)
        def _():
            pltpu.sync_copy(z128_hbm.at[pl.ds(9600, 400)], acc.at[pl.ds(9600, 400)])

        plsc.subcore_barrier()
        base = s * GPW2

        def run(pay_hbm):
            for b in range(2):
                off = base + b * GC
                pltpu.async_copy(cid_hbm.at[pl.ds(eoff + off, GC)],
                                 idx_v.at[b], si[b])
                pltpu.async_copy(pay_hbm.at[pl.ds(off, GC)], buf_v.at[b], sp[b])

            def body(it, carry):
                for b in range(2):
                    i = it * 2 + b
                    off = base + i * GC
                    pltpu.make_async_copy(cid_hbm.at[pl.ds(eoff + off, GC)],
                                          idx_v.at[b], si[b]).wait()
                    pltpu.make_async_copy(pay_hbm.at[pl.ds(off, GC)],
                                          buf_v.at[b], sp[b]).wait()
                    pltpu.sync_copy(buf_v.at[b], acc.at[idx_v.at[b]], add=True)

                    @pl.when(i + 2 < GNIT2)
                    def _():
                        pltpu.async_copy(
                            cid_hbm.at[pl.ds(eoff + off + 2 * GC, GC)],
                            idx_v.at[b], si[b])
                        pltpu.async_copy(pay_hbm.at[pl.ds(off + 2 * GC, GC)],
                                         buf_v.at[b], sp[b])
                return carry

            lax.fori_loop(0, GNIT2 // 2, body, 0)

        @pl.when(c == 0)
        def _():
            run(s128_hbm)

        @pl.when(c == 1)
        def _():
            run(e128_hbm)

        plsc.subcore_barrier()

        @pl.when(s < 15)
        def _():
            pltpu.sync_copy(acc.at[pl.ds(r0, 640)],
                            o_hbm.at[pl.ds(c * NN + r0, 640)])

        @pl.when(s == 15)
        def _():
            pltpu.sync_copy(acc.at[pl.ds(9600, 400)],
                            o_hbm.at[pl.ds(c * NN + 9600, 400)])

    return k(S128, E128, cid, z128)


def _attn_node(PA, PB, hV, lp):
    def body(pa, pb, hv, wo, g_, b_, out):
        a = pa[...]
        b2 = pb[...]
        numer = a[0:NN] + b2[0:NN]
        den = a[NN:2 * NN] + b2[NN:2 * NN] + 1e-16
        dh = _dot(numer / den, wo[...])
        out[...] = _bn_full(hv[...] + dh, g_[...], b_[...])

    return _pc(body, out_shape=jax.ShapeDtypeStruct((NN, HID), _F32))(
        PA, PB, hV, lp['WO'], _r1(lp['n0_g']), _r1(lp['n0_b']))


def _ffn_node(hV1, lp, res=None):
    def body(hv, w1, b1_, w2, b2_, g_, b_, *rest):
        x = hv[...]
        h = jnp.maximum(_dot(x, w1[...]) + b1_[...], 0.0)
        y = x + _dot(h, w2[...]) + b2_[...]
        z = _bn_full(y, g_[...], b_[...])
        if res is not None:
            z = z + rest[0][...]
        rest[-1][...] = z

    args = [hV1, lp['D1_W'], _r1(lp['D1_b']), lp['D2_W'], _r1(lp['D2_b']),
            _r1(lp['n1_g']), _r1(lp['n1_b'])]
    if res is not None:
        args.append(res)
    return _pc(body, out_shape=jax.ShapeDtypeStruct((NN, HID), _F32))(*args)


def _convk(x, W5, b):
    z2 = jnp.zeros((2, x.shape[1]), _F32)
    xp = jnp.concatenate([z2, x, z2], 0)
    acc = jnp.zeros((x.shape[0], HID), _F32) + b
    for k in range(5):
        acc = acc + _dot(xp[k:k + x.shape[0]], W5[k])
    return acc


def _cnn_in(x, w5a, ba, g1_, bb1, w5b, bb_, g2_, bb2, w5c, bc):
    y = jnp.maximum(_bn_full(_convk(x, w5a, ba), g1_, bb1), 0.0)
    y = jnp.maximum(_bn_full(_convk(y, w5b, bb_), g2_, bb2), 0.0)
    return _convk(y, w5c, bc)


def _cw(W):
    return jnp.transpose(W, (2, 1, 0))


def _dec1(hV, p):
    def body(hv, w5a, ba, g1_, bb1, w5b, bb_, g2_, bb2, w5c, bc, rW, rb,
             lp0_out, lg_out):
        h = _cnn_in(hv[...], w5a[...], ba[...], g1_[...], bb1[...], w5b[...],
                    bb_[...], g2_[...], bb2[...], w5c[...], bc[...])
        logits = _dot(h, rW[...]) + rb[...]
        m = jnp.max(logits, -1, keepdims=True)
        lse = m + jnp.log(jnp.sum(jnp.exp(logits - m), -1, keepdims=True))
        lp0_out[...] = logits - lse
        lg_out[...] = logits

    return _pc(body, out_shape=[jax.ShapeDtypeStruct((NN, 20), _F32),
                                jax.ShapeDtypeStruct((NN, 20), _F32)])(
        hV, _cw(p['c1_W']), _r1(p['c1_b']), _r1(p['bn1_g']), _r1(p['bn1_b']),
        _cw(p['c2_W']), _r1(p['c2_b']), _r1(p['bn2_g']), _r1(p['bn2_b']),
        _cw(p['c3_W']), _r1(p['c3_b']), p['r_W'], _r1(p['r_b']))


def _dec2(hV, logits, p):
    def body(hv, lg, tab, w5a, ba, g1_, bb1, w5b, bb_, g2_, bb2, w5c, bc,
             rW, rb, out):
        lgv = lg[...]
        v0 = jnp.max(lgv, -1, keepdims=True)
        cnt = jnp.sum(jnp.where(lgv == v0, 1.0, 0.0), -1, keepdims=True)
        v1m = jnp.max(jnp.where(lgv < v0, lgv, -jnp.inf), -1, keepdims=True)
        v1 = jnp.where(cnt > 1.0, v0, v1m)
        conf = jnp.clip((v0 / (v1 + 1e-5)).astype(jnp.int32), 0, 49)
        io50 = lax.broadcasted_iota(jnp.int32, (NN, 50), 1)
        oh = jnp.where(io50 == conf, 1.0, 0.0).astype(_F32)
        hC = _dot(oh, tab[...])
        x = jnp.concatenate([hv[...], hC], 1)
        h = _cnn_in(x, w5a[...], ba[...], g1_[...], bb1[...], w5b[...],
                    bb_[...], g2_[...], bb2[...], w5c[...], bc[...])
        lg2 = _dot(h, rW[...]) + rb[...]
        m = jnp.max(lg2, -1, keepdims=True)
        lse = m + jnp.log(jnp.sum(jnp.exp(lg2 - m), -1, keepdims=True))
        out[...] = lg2 - lse

    return _pc(body, out_shape=jax.ShapeDtypeStruct((NN, 20), _F32))(
        hV, logits, p['tab'],
        _cw(p['c1_W']), _r1(p['c1_b']), _r1(p['bn1_g']), _r1(p['bn1_b']),
        _cw(p['c2_W']), _r1(p['c2_b']), _r1(p['bn2_g']), _r1(p['bn2_b']),
        _cw(p['c3_W']), _r1(p['c3_b']), p['r_W'], _r1(p['r_b']))


def kernel(h_V, h_P, params, P_idx, batch_id):
    p = params
    cid = P_idx[0]
    src = P_idx[1]

    hV = _node_head(h_V, p)

    h_Pp = jnp.pad(h_P, ((0, 0), (0, 1)))
    eWp = jnp.pad(p['edge_W'], ((0, 1), (0, 0)))
    stats = _edge_stats(h_Pp, eWp, _r1(p['edge_b']))
    hP = _edge_embed(h_Pp, stats, eWp, _r1(p['edge_b']), _r1(p['ne_g']),
                     _r1(p['ne_b']), p['we_W'], _r1(p['we_b']))

    z128 = jnp.zeros((NN, 128), _F32)

    nblk2 = NE2 // EB
    for lp in p['enc']:
        for sub, resid in (('l1', False), ('l2', True)):
            lpp = lp[sub]
            gA1, gA2 = _sc_gather(hV, cid, src, 0)
            SA, EA = _edge_attn(gA1, gA2, hP, lpp, 0)
            gB1, gB2 = _sc_gather(hV, cid, src, NE2)
            SB, EBr = _edge_attn(gB1, gB2, hP, lpp, nblk2)
            PA = _sc_scatter(SA, EA, cid, z128, 0)
            PB = _sc_scatter(SB, EBr, cid, z128, NE2)
            hmid = _attn_node(PA, PB, hV, lpp)
            hV = _ffn_node(hmid, lpp, res=hV if resid else None)

    lp0, logits = _dec1(hV, p['dec1'])
    lp1 = _dec2(hV, logits, p['dec2'])
    return lp1, lp0


# speedup vs baseline: 1.0723x; 1.0723x over previous
import functools

import numpy as np
import jax
import jax.numpy as jnp
from jax import lax
from jax.experimental import pallas as pl
from jax.experimental.pallas import tpu as pltpu
from jax.experimental.pallas import tpu_sc as plsc

NN = 10000
NE = 320000
HID = 128

_F32 = jnp.float32
_pc = pl.pallas_call
EB = 2000


def _dot(a, b):
    return jnp.dot(a, b, preferred_element_type=_F32)


def _bn_full(x, g, b):
    m = jnp.mean(x, 0, keepdims=True)
    v = jnp.mean((x - m) ** 2, 0, keepdims=True)
    return (x - m) / jnp.sqrt(v + 1e-5) * g + b


def _r1(v):
    return v.reshape(1, -1)


def _node_head(h_V, p):
    def body(hv, nW, nb, nng, nnb, w1, b1, g1, bb1, w2, b2, g2, bb2, w3, b3, out):
        y = _dot(hv[...], nW[...]) + nb[...]
        y = _bn_full(y, nng[...], nnb[...])
        y = _dot(y, w1[...]) + b1[...]
        y = jnp.where(y >= 0, y, 0.01 * y)
        y = _bn_full(y, g1[...], bb1[...])
        y = _dot(y, w2[...]) + b2[...]
        y = jnp.where(y >= 0, y, 0.01 * y)
        y = _bn_full(y, g2[...], bb2[...])
        out[...] = _dot(y, w3[...]) + b3[...]

    hvp = jnp.pad(h_V, ((0, 0), (0, 4)))
    nWp = jnp.pad(p['node_W'], ((0, 4), (0, 0)))
    return _pc(body, out_shape=jax.ShapeDtypeStruct((NN, HID), _F32))(
        hvp, nWp, _r1(p['node_b']), _r1(p['nn_g']), _r1(p['nn_b']),
        p['wv1_W'], _r1(p['wv1_b']), _r1(p['wbn1_g']), _r1(p['wbn1_b']),
        p['wv2_W'], _r1(p['wv2_b']), _r1(p['wbn2_g']), _r1(p['wbn2_b']),
        p['wv3_W'], _r1(p['wv3_b']))


def _edge_stats(h_Pp, Wp, b):
    nblk = NE // EB

    def body(hp, W, bb, out, acc):
        i = pl.program_id(0)
        y = _dot(hp[...], W[...]) + bb[...]

        @pl.when(i == 0)
        def _():
            acc[...] = jnp.zeros_like(acc)

        acc[0:1, :] = acc[0:1, :] + jnp.sum(y, 0, keepdims=True)
        acc[1:2, :] = acc[1:2, :] + jnp.sum(y * y, 0, keepdims=True)
        out[...] = acc[...]

    return _pc(
        body,
        grid=(nblk,),
        in_specs=[pl.BlockSpec((EB, 24), lambda i: (i, 0)),
                  pl.BlockSpec((24, 128), lambda i: (0, 0)),
                  pl.BlockSpec((1, 128), lambda i: (0, 0))],
        out_specs=pl.BlockSpec((2, 128), lambda i: (0, 0)),
        out_shape=jax.ShapeDtypeStruct((2, 128), _F32),
        scratch_shapes=[pltpu.VMEM((2, 128), _F32)],
    )(h_Pp, Wp, b)


def _edge_embed(h_Pp, stats, Wp, b, g, bb, weW, web):
    nblk = NE // EB

    def body(hp, st, W, b_, g_, bb_, wW, wb, out):
        y = _dot(hp[...], W[...]) + b_[...]
        sa = st[...]
        m = sa[0:1, :] * np.float32(1.0 / NE)
        v = sa[1:2, :] * np.float32(1.0 / NE) - m * m
        y = (y - m) / jnp.sqrt(v + 1e-5) * g_[...] + bb_[...]
        out[...] = (_dot(y, wW[...]) + wb[...]).astype(jnp.bfloat16)

    return _pc(
        body,
        grid=(nblk,),
        in_specs=[pl.BlockSpec((EB, 24), lambda i: (i, 0)),
                  pl.BlockSpec((2, 128), lambda i: (0, 0)),
                  pl.BlockSpec((24, 128), lambda i: (0, 0)),
                  pl.BlockSpec((1, 128), lambda i: (0, 0)),
                  pl.BlockSpec((1, 128), lambda i: (0, 0)),
                  pl.BlockSpec((1, 128), lambda i: (0, 0)),
                  pl.BlockSpec((128, 128), lambda i: (0, 0)),
                  pl.BlockSpec((1, 128), lambda i: (0, 0))],
        out_specs=pl.BlockSpec((EB, 128), lambda i: (i, 0)),
        out_shape=jax.ShapeDtypeStruct((NE, 128), jnp.bfloat16),
    )(h_Pp, stats, Wp, b, g, bb, weW, web)


def _edge_attn(g1, g2, hP, lp):
    B1 = lp['B1_W']
    B3p = jnp.pad(lp['B3_W'], ((0, 0), (0, 12)))
    b3p = _r1(jnp.pad(lp['B3_b'], (0, 12)))
    WV = lp['WV']
    sq = np.float32(np.sqrt(32.0))
    nblk = NE // EB

    def body(g1r, g2r, hpr, a_, b_, c_, b1r, B2r, b2r, B3r, b3r, wva, wvb,
             o128, o8):
        x1 = g1r[...]
        x2 = g2r[...]
        xp = hpr[...].astype(_F32)
        h = _dot(x1, a_[...]) + _dot(xp, b_[...]) + _dot(x2, c_[...]) + b1r[...]
        h = jnp.maximum(h, 0.0)
        h = jnp.maximum(_dot(h, B2r[...]) + b2r[...], 0.0)
        w = (_dot(h, B3r[...]) + b3r[...]) / sq
        ex = jnp.exp(w)
        V = _dot(xp, wva[...]) + _dot(x2, wvb[...])
        row = lax.broadcasted_iota(jnp.int32, (16, 128), 0)
        col = lax.broadcasted_iota(jnp.int32, (16, 128), 1)
        R = jnp.where((col // 32 == row) & (row < 4), 1.0, 0.0).astype(_F32)
        E = _dot(ex, R)
        o128[...] = V * E
        o8[...] = E

    wspec = lambda shape: pl.BlockSpec(shape, lambda i: (0, 0))
    return _pc(
        body,
        grid=(nblk,),
        in_specs=[pl.BlockSpec((EB, 128), lambda i: (i, 0))] * 3 + [
            wspec((128, 128)), wspec((128, 128)), wspec((128, 128)),
            wspec((1, 128)), wspec((128, 128)), wspec((1, 128)),
            wspec((128, 16)), wspec((1, 16)),
            wspec((128, 128)), wspec((128, 128))],
        out_specs=[pl.BlockSpec((EB, 128), lambda i: (i, 0)),
                   pl.BlockSpec((EB, 128), lambda i: (i, 0))],
        out_shape=[jax.ShapeDtypeStruct((NE, 128), _F32),
                   jax.ShapeDtypeStruct((NE, 128), _F32)],
    )(g1, g2, hP,
      B1[0:128], B1[128:256], B1[256:384], _r1(lp['B1_b']),
      lp['B2_W'], _r1(lp['B2_b']), B3p, b3p, WV[0:128], WV[128:256])


_MESH = plsc.VectorSubcoreMesh(core_axis_name="c", subcore_axis_name="s")
GC = 80
GPW2 = NE // 16
GNIT2 = GPW2 // GC


def _sc_gather(table, cid, src):
    @functools.partial(
        pl.kernel,
        out_type=(jax.ShapeDtypeStruct((NE, 128), _F32),
                  jax.ShapeDtypeStruct((NE, 128), _F32)),
        mesh=_MESH,
        scratch_types=[pltpu.VMEM((2, GC), jnp.int32),
                       pltpu.VMEM((2, GC, 128), _F32),
                       pltpu.SemaphoreType.DMA, pltpu.SemaphoreType.DMA,
                       pltpu.SemaphoreType.DMA, pltpu.SemaphoreType.DMA,
                       pltpu.SemaphoreType.DMA, pltpu.SemaphoreType.DMA],
    )
    def k(tab_hbm, cid_hbm, src_hbm, o1_hbm, o2_hbm, idx_v, rows_v,
          si0, si1, sg0, sg1, sw0, sw1):
        c = lax.axis_index("c")
        s = lax.axis_index("s")
        base = s * GPW2
        si = (si0, si1)
        sg = (sg0, sg1)
        sw = (sw0, sw1)

        def run(idx_hbm, out_hbm):
            for b in range(2):
                pltpu.async_copy(idx_hbm.at[pl.ds(base + b * GC, GC)],
                                 idx_v.at[b], si[b])

            def body(it, carry):
                for b in range(2):
                    i = it * 2 + b
                    off = base + i * GC
                    pltpu.make_async_copy(idx_hbm.at[pl.ds(off, GC)],
                                          idx_v.at[b], si[b]).wait()

                    @pl.when(it > 0)
                    def _():
                        pltpu.make_async_copy(
                            rows_v.at[b],
                            out_hbm.at[pl.ds(off - 2 * GC, GC)], sw[b]).wait()

                    pltpu.async_copy(tab_hbm.at[idx_v.at[b]], rows_v.at[b],
                                     sg[b]).wait()
                    pltpu.async_copy(rows_v.at[b], out_hbm.at[pl.ds(off, GC)],
                                     sw[b])

                    @pl.when(i + 2 < GNIT2)
                    def _():
                        pltpu.async_copy(idx_hbm.at[pl.ds(off + 2 * GC, GC)],
                                         idx_v.at[b], si[b])
                return carry

            lax.fori_loop(0, GNIT2 // 2, body, 0)
            for b in range(2):
                off = base + (GNIT2 - 2 + b) * GC
                pltpu.make_async_copy(rows_v.at[b], out_hbm.at[pl.ds(off, GC)],
                                      sw[b]).wait()

        @pl.when(c == 0)
        def _():
            run(cid_hbm, o1_hbm)

        @pl.when(c == 1)
        def _():
            run(src_hbm, o2_hbm)

    return k(table, cid, src)


def _sc_scatter(S128, E128, cid, z128):
    @functools.partial(
        pl.kernel,
        out_type=jax.ShapeDtypeStruct((2 * NN, 128), _F32),
        mesh=_MESH,
        scratch_types=[pltpu.VMEM((2, GC), jnp.int32),
                       pltpu.VMEM((2, GC, 128), _F32),
                       pltpu.SemaphoreType.DMA, pltpu.SemaphoreType.DMA,
                       pltpu.SemaphoreType.DMA, pltpu.SemaphoreType.DMA,
                       pltpu.VMEM_SHARED((NN, 128), _F32)],
    )
    def k(s128_hbm, e128_hbm, cid_hbm, z128_hbm, o_hbm, idx_v, buf_v,
          si0, si1, sp0, sp1, acc):
        c = lax.axis_index("c")
        s = lax.axis_index("s")
        r0 = s * 640
        si = (si0, si1)
        sp = (sp0, sp1)

        @pl.when(s < 15)
        def _():
            pltpu.sync_copy(z128_hbm.at[pl.ds(r0, 640)], acc.at[pl.ds(r0, 640)])

        @pl.when(s == 15)
        def _():
            pltpu.sync_copy(z128_hbm.at[pl.ds(9600, 400)], acc.at[pl.ds(9600, 400)])

        plsc.subcore_barrier()
        base = s * GPW2

        def run(pay_hbm):
            for b in range(2):
                off = base + b * GC
                pltpu.async_copy(cid_hbm.at[pl.ds(off, GC)], idx_v.at[b], si[b])
                pltpu.async_copy(pay_hbm.at[pl.ds(off, GC)], buf_v.at[b], sp[b])

            def body(it, carry):
                for b in range(2):
                    i = it * 2 + b
                    off = base + i * GC
                    pltpu.make_async_copy(cid_hbm.at[pl.ds(off, GC)],
                                          idx_v.at[b], si[b]).wait()
                    pltpu.make_async_copy(pay_hbm.at[pl.ds(off, GC)],
                                          buf_v.at[b], sp[b]).wait()
                    pltpu.sync_copy(buf_v.at[b], acc.at[idx_v.at[b]], add=True)

                    @pl.when(i + 2 < GNIT2)
                    def _():
                        pltpu.async_copy(cid_hbm.at[pl.ds(off + 2 * GC, GC)],
                                         idx_v.at[b], si[b])
                        pltpu.async_copy(pay_hbm.at[pl.ds(off + 2 * GC, GC)],
                                         buf_v.at[b], sp[b])
                return carry

            lax.fori_loop(0, GNIT2 // 2, body, 0)

        @pl.when(c == 0)
        def _():
            run(s128_hbm)

        @pl.when(c == 1)
        def _():
            run(e128_hbm)

        plsc.subcore_barrier()

        @pl.when(s < 15)
        def _():
            pltpu.sync_copy(acc.at[pl.ds(r0, 640)],
                            o_hbm.at[pl.ds(c * NN + r0, 640)])

        @pl.when(s == 15)
        def _():
            pltpu.sync_copy(acc.at[pl.ds(9600, 400)],
                            o_hbm.at[pl.ds(c * NN + 9600, 400)])

    return k(S128, E128, cid, z128)


def _attn_node(P, hV, lp):
    def body(p_, hv, wo, g_, b_, out):
        a = p_[...]
        numer = a[0:NN]
        den = a[NN:2 * NN] + 1e-16
        dh = _dot(numer / den, wo[...])
        out[...] = _bn_full(hv[...] + dh, g_[...], b_[...])

    return _pc(body, out_shape=jax.ShapeDtypeStruct((NN, HID), _F32))(
        P, hV, lp['WO'], _r1(lp['n0_g']), _r1(lp['n0_b']))


def _ffn_node(hV1, lp, res=None):
    def body(hv, w1, b1_, w2, b2_, g_, b_, *rest):
        x = hv[...]
        h = jnp.maximum(_dot(x, w1[...]) + b1_[...], 0.0)
        y = x + _dot(h, w2[...]) + b2_[...]
        z = _bn_full(y, g_[...], b_[...])
        if res is not None:
            z = z + rest[0][...]
        rest[-1][...] = z

    args = [hV1, lp['D1_W'], _r1(lp['D1_b']), lp['D2_W'], _r1(lp['D2_b']),
            _r1(lp['n1_g']), _r1(lp['n1_b'])]
    if res is not None:
        args.append(res)
    return _pc(body, out_shape=jax.ShapeDtypeStruct((NN, HID), _F32))(*args)


def _convk(x, W5, b):
    z2 = jnp.zeros((2, x.shape[1]), _F32)
    xp = jnp.concatenate([z2, x, z2], 0)
    acc = jnp.zeros((x.shape[0], HID), _F32) + b
    for k in range(5):
        acc = acc + _dot(xp[k:k + x.shape[0]], W5[k])
    return acc


def _cnn_in(x, w5a, ba, g1_, bb1, w5b, bb_, g2_, bb2, w5c, bc):
    y = jnp.maximum(_bn_full(_convk(x, w5a, ba), g1_, bb1), 0.0)
    y = jnp.maximum(_bn_full(_convk(y, w5b, bb_), g2_, bb2), 0.0)
    return _convk(y, w5c, bc)


def _cw(W):
    return jnp.transpose(W, (2, 1, 0))


def _dec1(hV, p):
    def body(hv, w5a, ba, g1_, bb1, w5b, bb_, g2_, bb2, w5c, bc, rW, rb,
             lp0_out, lg_out):
        h = _cnn_in(hv[...], w5a[...], ba[...], g1_[...], bb1[...], w5b[...],
                    bb_[...], g2_[...], bb2[...], w5c[...], bc[...])
        logits = _dot(h, rW[...]) + rb[...]
        m = jnp.max(logits, -1, keepdims=True)
        lse = m + jnp.log(jnp.sum(jnp.exp(logits - m), -1, keepdims=True))
        lp0_out[...] = logits - lse
        lg_out[...] = logits

    return _pc(body, out_shape=[jax.ShapeDtypeStruct((NN, 20), _F32),
                                jax.ShapeDtypeStruct((NN, 20), _F32)])(
        hV, _cw(p['c1_W']), _r1(p['c1_b']), _r1(p['bn1_g']), _r1(p['bn1_b']),
        _cw(p['c2_W']), _r1(p['c2_b']), _r1(p['bn2_g']), _r1(p['bn2_b']),
        _cw(p['c3_W']), _r1(p['c3_b']), p['r_W'], _r1(p['r_b']))


def _dec2(hV, logits, p):
    def body(hv, lg, tab, w5a, ba, g1_, bb1, w5b, bb_, g2_, bb2, w5c, bc,
             rW, rb, out):
        lgv = lg[...]
        v0 = jnp.max(lgv, -1, keepdims=True)
        cnt = jnp.sum(jnp.where(lgv == v0, 1.0, 0.0), -1, keepdims=True)
        v1m = jnp.max(jnp.where(lgv < v0, lgv, -jnp.inf), -1, keepdims=True)
        v1 = jnp.where(cnt > 1.0, v0, v1m)
        conf = jnp.clip((v0 / (v1 + 1e-5)).astype(jnp.int32), 0, 49)
        io50 = lax.broadcasted_iota(jnp.int32, (NN, 50), 1)
        oh = jnp.where(io50 == conf, 1.0, 0.0).astype(_F32)
        hC = _dot(oh, tab[...])
        x = jnp.concatenate([hv[...], hC], 1)
        h = _cnn_in(x, w5a[...], ba[...], g1_[...], bb1[...], w5b[...],
                    bb_[...], g2_[...], bb2[...], w5c[...], bc[...])
        lg2 = _dot(h, rW[...]) + rb[...]
        m = jnp.max(lg2, -1, keepdims=True)
        lse = m + jnp.log(jnp.sum(jnp.exp(lg2 - m), -1, keepdims=True))
        out[...] = lg2 - lse

    return _pc(body, out_shape=jax.ShapeDtypeStruct((NN, 20), _F32))(
        hV, logits, p['tab'],
        _cw(p['c1_W']), _r1(p['c1_b']), _r1(p['bn1_g']), _r1(p['bn1_b']),
        _cw(p['c2_W']), _r1(p['c2_b']), _r1(p['bn2_g']), _r1(p['bn2_b']),
        _cw(p['c3_W']), _r1(p['c3_b']), p['r_W'], _r1(p['r_b']))


def kernel(h_V, h_P, params, P_idx, batch_id):
    p = params
    cid = P_idx[0]
    src = P_idx[1]

    hV = _node_head(h_V, p)

    h_Pp = jnp.pad(h_P, ((0, 0), (0, 1)))
    eWp = jnp.pad(p['edge_W'], ((0, 1), (0, 0)))
    stats = _edge_stats(h_Pp, eWp, _r1(p['edge_b']))
    hP = _edge_embed(h_Pp, stats, eWp, _r1(p['edge_b']), _r1(p['ne_g']),
                     _r1(p['ne_b']), p['we_W'], _r1(p['we_b']))

    z128 = jnp.zeros((NN, 128), _F32)

    for lp in p['enc']:
        for sub, resid in (('l1', False), ('l2', True)):
            lpp = lp[sub]
            g1, g2 = _sc_gather(hV, cid, src)
            S128, E128 = _edge_attn(g1, g2, hP, lpp)
            P = _sc_scatter(S128, E128, cid, z128)
            hmid = _attn_node(P, hV, lpp)
            hV = _ffn_node(hmid, lpp, res=hV if resid else None)

    lp0, logits = _dec1(hV, p['dec1'])
    lp1 = _dec2(hV, logits, p['dec2'])
    return lp1, lp0


# speedup vs baseline: 1.1512x; 1.0736x over previous
import functools

import numpy as np
import jax
import jax.numpy as jnp
from jax import lax
from jax.experimental import pallas as pl
from jax.experimental.pallas import tpu as pltpu
from jax.experimental.pallas import tpu_sc as plsc

NN = 10000
NE = 320000
HID = 128

_F32 = jnp.float32
_pc = pl.pallas_call
EB = 2000


def _dot(a, b):
    return jnp.dot(a, b, preferred_element_type=_F32)


def _bn_full(x, g, b):
    m = jnp.mean(x, 0, keepdims=True)
    v = jnp.mean((x - m) ** 2, 0, keepdims=True)
    return (x - m) / jnp.sqrt(v + 1e-5) * g + b


def _r1(v):
    return v.reshape(1, -1)


def _node_head(h_V, p):
    def body(hv, nW, nb, nng, nnb, w1, b1, g1, bb1, w2, b2, g2, bb2, w3, b3, out):
        y = _dot(hv[...], nW[...]) + nb[...]
        y = _bn_full(y, nng[...], nnb[...])
        y = _dot(y, w1[...]) + b1[...]
        y = jnp.where(y >= 0, y, 0.01 * y)
        y = _bn_full(y, g1[...], bb1[...])
        y = _dot(y, w2[...]) + b2[...]
        y = jnp.where(y >= 0, y, 0.01 * y)
        y = _bn_full(y, g2[...], bb2[...])
        out[...] = _dot(y, w3[...]) + b3[...]

    hvp = jnp.pad(h_V, ((0, 0), (0, 4)))
    nWp = jnp.pad(p['node_W'], ((0, 4), (0, 0)))
    return _pc(body, out_shape=jax.ShapeDtypeStruct((NN, HID), _F32))(
        hvp, nWp, _r1(p['node_b']), _r1(p['nn_g']), _r1(p['nn_b']),
        p['wv1_W'], _r1(p['wv1_b']), _r1(p['wbn1_g']), _r1(p['wbn1_b']),
        p['wv2_W'], _r1(p['wv2_b']), _r1(p['wbn2_g']), _r1(p['wbn2_b']),
        p['wv3_W'], _r1(p['wv3_b']))


def _edge_stats(h_Pp, Wp, b):
    nblk = NE // EB

    def body(hp, W, bb, out, acc):
        i = pl.program_id(0)
        y = _dot(hp[...], W[...]) + bb[...]

        @pl.when(i == 0)
        def _():
            acc[...] = jnp.zeros_like(acc)

        acc[0:1, :] = acc[0:1, :] + jnp.sum(y, 0, keepdims=True)
        acc[1:2, :] = acc[1:2, :] + jnp.sum(y * y, 0, keepdims=True)
        out[...] = acc[...]

    return _pc(
        body,
        grid=(nblk,),
        in_specs=[pl.BlockSpec((EB, 24), lambda i: (i, 0)),
                  pl.BlockSpec((24, 128), lambda i: (0, 0)),
                  pl.BlockSpec((1, 128), lambda i: (0, 0))],
        out_specs=pl.BlockSpec((2, 128), lambda i: (0, 0)),
        out_shape=jax.ShapeDtypeStruct((2, 128), _F32),
        scratch_shapes=[pltpu.VMEM((2, 128), _F32)],
    )(h_Pp, Wp, b)


def _edge_embed(h_Pp, stats, Wp, b, g, bb, weW, web):
    nblk = NE // EB

    def body(hp, st, W, b_, g_, bb_, wW, wb, out):
        y = _dot(hp[...], W[...]) + b_[...]
        sa = st[...]
        m = sa[0:1, :] * np.float32(1.0 / NE)
        v = sa[1:2, :] * np.float32(1.0 / NE) - m * m
        y = (y - m) / jnp.sqrt(v + 1e-5) * g_[...] + bb_[...]
        out[...] = (_dot(y, wW[...]) + wb[...]).astype(jnp.bfloat16)

    return _pc(
        body,
        grid=(nblk,),
        in_specs=[pl.BlockSpec((EB, 24), lambda i: (i, 0)),
                  pl.BlockSpec((2, 128), lambda i: (0, 0)),
                  pl.BlockSpec((24, 128), lambda i: (0, 0)),
                  pl.BlockSpec((1, 128), lambda i: (0, 0)),
                  pl.BlockSpec((1, 128), lambda i: (0, 0)),
                  pl.BlockSpec((1, 128), lambda i: (0, 0)),
                  pl.BlockSpec((128, 128), lambda i: (0, 0)),
                  pl.BlockSpec((1, 128), lambda i: (0, 0))],
        out_specs=pl.BlockSpec((EB, 128), lambda i: (i, 0)),
        out_shape=jax.ShapeDtypeStruct((NE, 128), jnp.bfloat16),
    )(h_Pp, stats, Wp, b, g, bb, weW, web)


def _edge_attn(g1, g2, hP, lp):
    B1 = lp['B1_W']
    B3p = jnp.pad(lp['B3_W'], ((0, 0), (0, 12)))
    b3p = _r1(jnp.pad(lp['B3_b'], (0, 12)))
    WV = lp['WV']
    sq = np.float32(np.sqrt(32.0))
    nblk = NE // EB

    def body(g1r, g2r, hpr, a_, b_, c_, b1r, B2r, b2r, B3r, b3r, wva, wvb,
             o128, o8):
        x1 = g1r[...]
        x2 = g2r[...]
        xp = hpr[...].astype(_F32)
        h = _dot(x1, a_[...]) + _dot(xp, b_[...]) + _dot(x2, c_[...]) + b1r[...]
        h = jnp.maximum(h, 0.0)
        h = jnp.maximum(_dot(h, B2r[...]) + b2r[...], 0.0)
        w = (_dot(h, B3r[...]) + b3r[...]) / sq
        ex = jnp.exp(w)
        V = _dot(xp, wva[...]) + _dot(x2, wvb[...])
        row = lax.broadcasted_iota(jnp.int32, (16, 128), 0)
        col = lax.broadcasted_iota(jnp.int32, (16, 128), 1)
        R = jnp.where((col // 32 == row) & (row < 4), 1.0, 0.0).astype(_F32)
        E = _dot(ex, R)
        o128[...] = V * E
        o8[...] = E

    wspec = lambda shape: pl.BlockSpec(shape, lambda i: (0, 0))
    return _pc(
        body,
        grid=(nblk,),
        in_specs=[pl.BlockSpec((EB, 128), lambda i: (i, 0))] * 3 + [
            wspec((128, 128)), wspec((128, 128)), wspec((128, 128)),
            wspec((1, 128)), wspec((128, 128)), wspec((1, 128)),
            wspec((128, 16)), wspec((1, 16)),
            wspec((128, 128)), wspec((128, 128))],
        out_specs=[pl.BlockSpec((EB, 128), lambda i: (i, 0)),
                   pl.BlockSpec((EB, 128), lambda i: (i, 0))],
        out_shape=[jax.ShapeDtypeStruct((NE, 128), _F32),
                   jax.ShapeDtypeStruct((NE, 128), _F32)],
    )(g1, g2, hP,
      B1[0:128], B1[128:256], B1[256:384], _r1(lp['B1_b']),
      lp['B2_W'], _r1(lp['B2_b']), B3p, b3p, WV[0:128], WV[128:256])


_MESH = plsc.VectorSubcoreMesh(core_axis_name="c", subcore_axis_name="s")
GC = 128
GPW2 = NE // 16
GNIT2 = 156
GTAIL = GPW2 - GNIT2 * GC


def _sc_gather(table, cid, src):
    @functools.partial(
        pl.kernel,
        out_type=(jax.ShapeDtypeStruct((NE, 128), _F32),
                  jax.ShapeDtypeStruct((NE, 128), _F32)),
        mesh=_MESH,
        scratch_types=[pltpu.VMEM((2, GC), jnp.int32),
                       pltpu.VMEM((2, GC, 128), _F32),
                       pltpu.VMEM((1, GTAIL), jnp.int32),
                       pltpu.VMEM((GTAIL, 128), _F32),
                       pltpu.SemaphoreType.DMA, pltpu.SemaphoreType.DMA,
                       pltpu.SemaphoreType.DMA, pltpu.SemaphoreType.DMA,
                       pltpu.SemaphoreType.DMA, pltpu.SemaphoreType.DMA],
    )
    def k(tab_hbm, cid_hbm, src_hbm, o1_hbm, o2_hbm, idx_v, rows_v,
          idxt_v, rowst_v, si0, si1, sg0, sg1, sw0, sw1):
        c = lax.axis_index("c")
        s = lax.axis_index("s")
        base = s * GPW2
        si = (si0, si1)
        sg = (sg0, sg1)
        sw = (sw0, sw1)

        def run(idx_hbm, out_hbm):
            for b in range(2):
                pltpu.async_copy(idx_hbm.at[pl.ds(base + b * GC, GC)],
                                 idx_v.at[b], si[b])

            def body(it, carry):
                for b in range(2):
                    i = it * 2 + b
                    off = base + i * GC
                    pltpu.make_async_copy(idx_hbm.at[pl.ds(off, GC)],
                                          idx_v.at[b], si[b]).wait()

                    @pl.when(it > 0)
                    def _():
                        pltpu.make_async_copy(
                            rows_v.at[b],
                            out_hbm.at[pl.ds(off - 2 * GC, GC)], sw[b]).wait()

                    pltpu.async_copy(tab_hbm.at[idx_v.at[b]], rows_v.at[b],
                                     sg[b]).wait()
                    pltpu.async_copy(rows_v.at[b], out_hbm.at[pl.ds(off, GC)],
                                     sw[b])

                    @pl.when(i + 2 < GNIT2)
                    def _():
                        pltpu.async_copy(idx_hbm.at[pl.ds(off + 2 * GC, GC)],
                                         idx_v.at[b], si[b])
                return carry

            lax.fori_loop(0, GNIT2 // 2, body, 0)
            offt = base + GNIT2 * GC
            pltpu.sync_copy(idx_hbm.at[pl.ds(offt, GTAIL)], idxt_v.at[0])
            pltpu.async_copy(tab_hbm.at[idxt_v.at[0]], rowst_v, sg[0]).wait()
            pltpu.async_copy(rowst_v, out_hbm.at[pl.ds(offt, GTAIL)], sw[0])
            for b in range(2):
                off = base + (GNIT2 - 2 + b) * GC
                pltpu.make_async_copy(rows_v.at[b], out_hbm.at[pl.ds(off, GC)],
                                      sw[b]).wait()
            pltpu.make_async_copy(rowst_v, out_hbm.at[pl.ds(offt, GTAIL)],
                                  sw[0]).wait()

        @pl.when(c == 0)
        def _():
            run(cid_hbm, o1_hbm)

        @pl.when(c == 1)
        def _():
            run(src_hbm, o2_hbm)

    return k(table, cid, src)


def _sc_scatter(S128, E128, cid, z128):
    @functools.partial(
        pl.kernel,
        out_type=jax.ShapeDtypeStruct((2 * NN, 128), _F32),
        mesh=_MESH,
        scratch_types=[pltpu.VMEM((2, GC), jnp.int32),
                       pltpu.VMEM((2, GC, 128), _F32),
                       pltpu.VMEM((1, GTAIL), jnp.int32),
                       pltpu.VMEM((GTAIL, 128), _F32),
                       pltpu.SemaphoreType.DMA, pltpu.SemaphoreType.DMA,
                       pltpu.SemaphoreType.DMA, pltpu.SemaphoreType.DMA,
                       pltpu.VMEM_SHARED((NN, 128), _F32)],
    )
    def k(s128_hbm, e128_hbm, cid_hbm, z128_hbm, o_hbm, idx_v, buf_v,
          idxt_v, buft_v, si0, si1, sp0, sp1, acc):
        c = lax.axis_index("c")
        s = lax.axis_index("s")
        r0 = s * 640
        si = (si0, si1)
        sp = (sp0, sp1)

        @pl.when(s < 15)
        def _():
            pltpu.sync_copy(z128_hbm.at[pl.ds(r0, 640)], acc.at[pl.ds(r0, 640)])

        @pl.when(s == 15)
        def _():
            pltpu.sync_copy(z128_hbm.at[pl.ds(9600, 400)], acc.at[pl.ds(9600, 400)])

        plsc.subcore_barrier()
        base = s * GPW2

        def run(pay_hbm):
            for b in range(2):
                off = base + b * GC
                pltpu.async_copy(cid_hbm.at[pl.ds(off, GC)], idx_v.at[b], si[b])
                pltpu.async_copy(pay_hbm.at[pl.ds(off, GC)], buf_v.at[b], sp[b])

            def body(it, carry):
                for b in range(2):
                    i = it * 2 + b
                    off = base + i * GC
                    pltpu.make_async_copy(cid_hbm.at[pl.ds(off, GC)],
                                          idx_v.at[b], si[b]).wait()
                    pltpu.make_async_copy(pay_hbm.at[pl.ds(off, GC)],
                                          buf_v.at[b], sp[b]).wait()
                    pltpu.sync_copy(buf_v.at[b], acc.at[idx_v.at[b]], add=True)

                    @pl.when(i + 2 < GNIT2)
                    def _():
                        pltpu.async_copy(cid_hbm.at[pl.ds(off + 2 * GC, GC)],
                                         idx_v.at[b], si[b])
                        pltpu.async_copy(pay_hbm.at[pl.ds(off + 2 * GC, GC)],
                                         buf_v.at[b], sp[b])
                return carry

            lax.fori_loop(0, GNIT2 // 2, body, 0)
            offt = base + GNIT2 * GC
            pltpu.sync_copy(cid_hbm.at[pl.ds(offt, GTAIL)], idxt_v.at[0])
            pltpu.sync_copy(pay_hbm.at[pl.ds(offt, GTAIL)], buft_v)
            pltpu.sync_copy(buft_v, acc.at[idxt_v.at[0]], add=True)

        @pl.when(c == 0)
        def _():
            run(s128_hbm)

        @pl.when(c == 1)
        def _():
            run(e128_hbm)

        plsc.subcore_barrier()

        @pl.when(s < 15)
        def _():
            pltpu.sync_copy(acc.at[pl.ds(r0, 640)],
                            o_hbm.at[pl.ds(c * NN + r0, 640)])

        @pl.when(s == 15)
        def _():
            pltpu.sync_copy(acc.at[pl.ds(9600, 400)],
                            o_hbm.at[pl.ds(c * NN + 9600, 400)])

    return k(S128, E128, cid, z128)


def _attn_node(P, hV, lp):
    def body(p_, hv, wo, g_, b_, out):
        a = p_[...]
        numer = a[0:NN]
        den = a[NN:2 * NN] + 1e-16
        dh = _dot(numer / den, wo[...])
        out[...] = _bn_full(hv[...] + dh, g_[...], b_[...])

    return _pc(body, out_shape=jax.ShapeDtypeStruct((NN, HID), _F32))(
        P, hV, lp['WO'], _r1(lp['n0_g']), _r1(lp['n0_b']))


def _ffn_node(hV1, lp, res=None):
    def body(hv, w1, b1_, w2, b2_, g_, b_, *rest):
        x = hv[...]
        h = jnp.maximum(_dot(x, w1[...]) + b1_[...], 0.0)
        y = x + _dot(h, w2[...]) + b2_[...]
        z = _bn_full(y, g_[...], b_[...])
        if res is not None:
            z = z + rest[0][...]
        rest[-1][...] = z

    args = [hV1, lp['D1_W'], _r1(lp['D1_b']), lp['D2_W'], _r1(lp['D2_b']),
            _r1(lp['n1_g']), _r1(lp['n1_b'])]
    if res is not None:
        args.append(res)
    return _pc(body, out_shape=jax.ShapeDtypeStruct((NN, HID), _F32))(*args)


def _convk(x, W5, b):
    z2 = jnp.zeros((2, x.shape[1]), _F32)
    xp = jnp.concatenate([z2, x, z2], 0)
    acc = jnp.zeros((x.shape[0], HID), _F32) + b
    for k in range(5):
        acc = acc + _dot(xp[k:k + x.shape[0]], W5[k])
    return acc


def _cnn_in(x, w5a, ba, g1_, bb1, w5b, bb_, g2_, bb2, w5c, bc):
    y = jnp.maximum(_bn_full(_convk(x, w5a, ba), g1_, bb1), 0.0)
    y = jnp.maximum(_bn_full(_convk(y, w5b, bb_), g2_, bb2), 0.0)
    return _convk(y, w5c, bc)


def _cw(W):
    return jnp.transpose(W, (2, 1, 0))


def _dec1(hV, p):
    def body(hv, w5a, ba, g1_, bb1, w5b, bb_, g2_, bb2, w5c, bc, rW, rb,
             lp0_out, lg_out):
        h = _cnn_in(hv[...], w5a[...], ba[...], g1_[...], bb1[...], w5b[...],
                    bb_[...], g2_[...], bb2[...], w5c[...], bc[...])
        logits = _dot(h, rW[...]) + rb[...]
        m = jnp.max(logits, -1, keepdims=True)
        lse = m + jnp.log(jnp.sum(jnp.exp(logits - m), -1, keepdims=True))
        lp0_out[...] = logits - lse
        lg_out[...] = logits

    return _pc(body, out_shape=[jax.ShapeDtypeStruct((NN, 20), _F32),
                                jax.ShapeDtypeStruct((NN, 20), _F32)])(
        hV, _cw(p['c1_W']), _r1(p['c1_b']), _r1(p['bn1_g']), _r1(p['bn1_b']),
        _cw(p['c2_W']), _r1(p['c2_b']), _r1(p['bn2_g']), _r1(p['bn2_b']),
        _cw(p['c3_W']), _r1(p['c3_b']), p['r_W'], _r1(p['r_b']))


def _dec2(hV, logits, p):
    def body(hv, lg, tab, w5a, ba, g1_, bb1, w5b, bb_, g2_, bb2, w5c, bc,
             rW, rb, out):
        lgv = lg[...]
        v0 = jnp.max(lgv, -1, keepdims=True)
        cnt = jnp.sum(jnp.where(lgv == v0, 1.0, 0.0), -1, keepdims=True)
        v1m = jnp.max(jnp.where(lgv < v0, lgv, -jnp.inf), -1, keepdims=True)
        v1 = jnp.where(cnt > 1.0, v0, v1m)
        conf = jnp.clip((v0 / (v1 + 1e-5)).astype(jnp.int32), 0, 49)
        io50 = lax.broadcasted_iota(jnp.int32, (NN, 50), 1)
        oh = jnp.where(io50 == conf, 1.0, 0.0).astype(_F32)
        hC = _dot(oh, tab[...])
        x = jnp.concatenate([hv[...], hC], 1)
        h = _cnn_in(x, w5a[...], ba[...], g1_[...], bb1[...], w5b[...],
                    bb_[...], g2_[...], bb2[...], w5c[...], bc[...])
        lg2 = _dot(h, rW[...]) + rb[...]
        m = jnp.max(lg2, -1, keepdims=True)
        lse = m + jnp.log(jnp.sum(jnp.exp(lg2 - m), -1, keepdims=True))
        out[...] = lg2 - lse

    return _pc(body, out_shape=jax.ShapeDtypeStruct((NN, 20), _F32))(
        hV, logits, p['tab'],
        _cw(p['c1_W']), _r1(p['c1_b']), _r1(p['bn1_g']), _r1(p['bn1_b']),
        _cw(p['c2_W']), _r1(p['c2_b']), _r1(p['bn2_g']), _r1(p['bn2_b']),
        _cw(p['c3_W']), _r1(p['c3_b']), p['r_W'], _r1(p['r_b']))


def kernel(h_V, h_P, params, P_idx, batch_id):
    p = params
    cid = P_idx[0]
    src = P_idx[1]

    hV = _node_head(h_V, p)

    h_Pp = jnp.pad(h_P, ((0, 0), (0, 1)))
    eWp = jnp.pad(p['edge_W'], ((0, 1), (0, 0)))
    stats = _edge_stats(h_Pp, eWp, _r1(p['edge_b']))
    hP = _edge_embed(h_Pp, stats, eWp, _r1(p['edge_b']), _r1(p['ne_g']),
                     _r1(p['ne_b']), p['we_W'], _r1(p['we_b']))

    z128 = jnp.zeros((NN, 128), _F32)

    for lp in p['enc']:
        for sub, resid in (('l1', False), ('l2', True)):
            lpp = lp[sub]
            g1, g2 = _sc_gather(hV, cid, src)
            S128, E128 = _edge_attn(g1, g2, hP, lpp)
            P = _sc_scatter(S128, E128, cid, z128)
            hmid = _attn_node(P, hV, lpp)
            hV = _ffn_node(hmid, lpp, res=hV if resid else None)

    lp0, logits = _dec1(hV, p['dec1'])
    lp1 = _dec2(hV, logits, p['dec2'])
    return lp1, lp0
